# Initial kernel scaffold; baseline (speedup 1.0000x reference)
import jax
import jax.numpy as jnp
from jax import lax
from jax.experimental import pallas as pl
from jax.experimental.pallas import tpu as pltpu
from jax.experimental.pallas import tpu_sc as plsc

_N_FIELDS = 26
_VOCAB = 100000
_EMB = 64
_BATCH = 4096
_NC, _NS, _L = 2, 16, 16
_NW = _NC * _NS
_KPW = _BATCH * _N_FIELDS // _NW
_CHUNK = 128
_NCH = _KPW // _CHUNK
_NSL = _KPW // _L


def _tile_body(x_hbm, tab_hbm, out_hbm, xv, idxv, rows0, rows1, sem0, sem1):
    wid = lax.axis_index("s") * _NC + lax.axis_index("c")
    base = wid * _KPW

    pltpu.sync_copy(x_hbm.at[pl.ds(base, _KPW)], xv)
    for s in range(_NSL):
        pos = lax.iota(jnp.int32, (_L,), 0) + (s * _L)
        fld = lax.rem(pos, _N_FIELDS)
        idxv[s * _L // _CHUNK, pl.ds(s * _L % _CHUNK, _L)] = (
            xv[pl.ds(s * _L, _L)] + fld * _VOCAB
        )

    bufs = (rows0, rows1)
    sems = (sem0, sem1)
    for j in range(_NCH):
        b = j % 2
        pltpu.async_copy(tab_hbm.at[idxv.at[j]], bufs[b], sems[b]).wait()
        pltpu.sync_copy(bufs[b], out_hbm.at[pl.ds(base + j * _CHUNK, _CHUNK)])


def kernel(x, tables):
    xf = x.reshape(_BATCH * _N_FIELDS)
    tf = tables.reshape(_N_FIELDS * _VOCAB, _EMB)
    mesh = plsc.VectorSubcoreMesh(core_axis_name="c", subcore_axis_name="s")
    out = pl.kernel(
        _tile_body,
        out_type=jax.ShapeDtypeStruct((_BATCH * _N_FIELDS, _EMB), jnp.float32),
        mesh=mesh,
        scratch_types=[
            pltpu.VMEM((_KPW,), jnp.int32),
            pltpu.VMEM((_NCH, _CHUNK), jnp.int32),
            pltpu.VMEM((_CHUNK, _EMB), jnp.float32),
            pltpu.VMEM((_CHUNK, _EMB), jnp.float32),
            pltpu.SemaphoreType.DMA,
            pltpu.SemaphoreType.DMA,
        ],
    )(xf, tf)
    return out.reshape(_BATCH, _N_FIELDS * _EMB)


# speedup vs baseline: 1.0382x; 1.0382x over previous
import jax
import jax.numpy as jnp
from jax import lax
from jax.experimental import pallas as pl
from jax.experimental.pallas import tpu as pltpu
from jax.experimental.pallas import tpu_sc as plsc

_N_FIELDS = 26
_VOCAB = 100000
_EMB = 64
_BATCH = 4096
_NC, _NS, _L = 2, 16, 16
_NW = _NC * _NS
_KPW = _BATCH * _N_FIELDS // _NW
_CHUNK = 128
_NCH = _KPW // _CHUNK
_NSL = _KPW // _L


def _tile_body(x_hbm, tab_hbm, out_hbm, xv, idxv, rows0, rows1, sem0, sem1):
    wid = lax.axis_index("s") * _NC + lax.axis_index("c")
    base = wid * _KPW

    pltpu.sync_copy(x_hbm.at[pl.ds(base, _KPW)], xv)
    for s in range(_NSL):
        pos = lax.iota(jnp.int32, _L) + (s * _L)
        fld = lax.rem(pos, _N_FIELDS)
        idxv[s * _L // _CHUNK, pl.ds(s * _L % _CHUNK, _L)] = (
            xv[pl.ds(s * _L, _L)] + fld * _VOCAB
        )

    bufs = (rows0, rows1)
    sems = (sem0, sem1)
    for j in range(_NCH):
        b = j % 2
        pltpu.async_copy(tab_hbm.at[idxv.at[j]], bufs[b], sems[b]).wait()
        pltpu.sync_copy(bufs[b], out_hbm.at[pl.ds(base + j * _CHUNK, _CHUNK)])


def kernel(x, tables):
    xf = x.reshape(_BATCH * _N_FIELDS)
    tf = tables.reshape(_N_FIELDS * _VOCAB, _EMB)
    mesh = plsc.VectorSubcoreMesh(core_axis_name="c", subcore_axis_name="s")
    out = pl.kernel(
        _tile_body,
        out_type=jax.ShapeDtypeStruct((_BATCH * _N_FIELDS, _EMB), jnp.float32),
        mesh=mesh,
        compiler_params=pltpu.CompilerParams(use_tc_tiling_on_sc=False),
        scratch_types=[
            pltpu.VMEM((_KPW,), jnp.int32),
            pltpu.VMEM((_NCH, _CHUNK), jnp.int32),
            pltpu.VMEM((_CHUNK, _EMB), jnp.float32),
            pltpu.VMEM((_CHUNK, _EMB), jnp.float32),
            pltpu.SemaphoreType.DMA,
            pltpu.SemaphoreType.DMA,
        ],
    )(xf, tf)
    return out.reshape(_BATCH, _N_FIELDS * _EMB)


# speedup vs baseline: 1.0509x; 1.0122x over previous
import jax
import jax.numpy as jnp
from jax import lax
from jax.experimental import pallas as pl
from jax.experimental.pallas import tpu as pltpu
from jax.experimental.pallas import tpu_sc as plsc

_N_FIELDS = 26
_VOCAB = 100000
_EMB = 64
_BATCH = 4096
_NC, _NS, _L = 2, 16, 16
_NW = _NC * _NS
_KPW = _BATCH * _N_FIELDS // _NW
_CHUNK = 128
_NCH = _KPW // _CHUNK
_NSL = _KPW // _L


_NBUF = 4


def _tile_body(x_hbm, tab_hbm, out_hbm, xv, idxv, *rest):
    bufs = rest[:_NBUF]
    gsems = rest[_NBUF:2 * _NBUF]
    ssems = rest[2 * _NBUF:3 * _NBUF]
    wid = lax.axis_index("s") * _NC + lax.axis_index("c")
    base = wid * _KPW

    pltpu.sync_copy(x_hbm.at[pl.ds(base, _KPW)], xv)
    for s in range(_NSL):
        pos = lax.iota(jnp.int32, _L) + (s * _L)
        fld = lax.rem(pos, _N_FIELDS)
        idxv[s * _L // _CHUNK, pl.ds(s * _L % _CHUNK, _L)] = (
            xv[pl.ds(s * _L, _L)] + fld * _VOCAB
        )

    def _gather(j, b):
        return pltpu.async_copy(tab_hbm.at[idxv.at[j]], bufs[b], gsems[b])

    def _store(j, b):
        return pltpu.async_copy(
            bufs[b], out_hbm.at[pl.ds(base + j * _CHUNK, _CHUNK)], ssems[b]
        )

    gh, sh = {}, {}
    unwaited_stores = set()
    for j in range(min(_NBUF - 1, _NCH)):
        gh[j] = _gather(j, j)
    for j in range(_NCH):
        gh[j].wait()
        sh[j] = _store(j, j % _NBUF)
        unwaited_stores.add(j)
        nj = j + _NBUF - 1
        if nj < _NCH:
            pj = nj - _NBUF
            if pj >= 0:
                sh[pj].wait()
                unwaited_stores.discard(pj)
            gh[nj] = _gather(nj, nj % _NBUF)
    for j in sorted(unwaited_stores):
        sh[j].wait()


def kernel(x, tables):
    xf = x.reshape(_BATCH * _N_FIELDS)
    tf = tables.reshape(_N_FIELDS * _VOCAB, _EMB)
    mesh = plsc.VectorSubcoreMesh(core_axis_name="c", subcore_axis_name="s")
    out = pl.kernel(
        _tile_body,
        out_type=jax.ShapeDtypeStruct((_BATCH * _N_FIELDS, _EMB), jnp.float32),
        mesh=mesh,
        compiler_params=pltpu.CompilerParams(use_tc_tiling_on_sc=False),
        scratch_types=(
            [
                pltpu.VMEM((_KPW,), jnp.int32),
                pltpu.VMEM((_NCH, _CHUNK), jnp.int32),
            ]
            + [pltpu.VMEM((_CHUNK, _EMB), jnp.float32) for _ in range(_NBUF)]
            + [pltpu.SemaphoreType.DMA for _ in range(2 * _NBUF)]
        ),
    )(xf, tf)
    return out.reshape(_BATCH, _N_FIELDS * _EMB)


# speedup vs baseline: 4.6476x; 4.4226x over previous
import jax
import jax.numpy as jnp
from jax import lax
from jax.experimental import pallas as pl
from jax.experimental.pallas import tpu as pltpu
from jax.experimental.pallas import tpu_sc as plsc

_N_FIELDS = 26
_VOCAB = 100000
_EMB = 64
_BATCH = 4096
_NC, _NS, _L = 2, 16, 16
_NW = _NC * _NS
_NTASK = _N_FIELDS * _EMB
_TPW = _NTASK // _NW
_NG = _BATCH // _L


def _tile_body(xT_hbm, tt_hbm, outT_hbm, tv, xv0, xv1, ov0, ov1,
               tsem, xsem0, xsem1, ssem0, ssem1):
    wid = lax.axis_index("s") * _NC + lax.axis_index("c")
    xvs, ovs = (xv0, xv1), (ov0, ov1)
    xsems, ssems = (xsem0, xsem1), (ssem0, ssem1)

    def body(j, _):
        for sub in range(2):
            t = wid * _TPW + 2 * j + sub
            i = t // _EMB
            c = t % _EMB
            hx = pltpu.make_async_copy(xT_hbm.at[i], xvs[sub], xsems[sub])
            hx.start()
            ht = pltpu.make_async_copy(tt_hbm.at[i, c], tv, tsem)
            ht.start()
            @pl.when(j >= 1)
            def _():
                pltpu.make_async_copy(ovs[sub], outT_hbm.at[t], ssems[sub]).wait()
            hx.wait()
            ht.wait()
            def gather(g, _):
                for u in range(16):
                    off = (g * 16 + u) * _L
                    idx = xvs[sub][pl.ds(off, _L)]
                    ovs[sub][pl.ds(off, _L)] = plsc.load_gather(tv, [idx])
                return 0
            lax.fori_loop(0, _NG // 16, gather, 0)
            pltpu.make_async_copy(ovs[sub], outT_hbm.at[t], ssems[sub]).start()
        return 0

    lax.fori_loop(0, _TPW // 2, body, 0)
    last = wid * _TPW + _TPW - 2
    pltpu.make_async_copy(ov0, outT_hbm.at[last], ssem0).wait()
    pltpu.make_async_copy(ov1, outT_hbm.at[last + 1], ssem1).wait()


def kernel(x, tables):
    xT = jnp.transpose(x)
    tt = jnp.transpose(tables, (0, 2, 1))
    mesh = plsc.VectorSubcoreMesh(core_axis_name="c", subcore_axis_name="s")
    outT = pl.kernel(
        _tile_body,
        out_type=jax.ShapeDtypeStruct((_NTASK, _BATCH), jnp.float32),
        mesh=mesh,
        compiler_params=pltpu.CompilerParams(needs_layout_passes=False),
        scratch_types=[
            pltpu.VMEM((_VOCAB,), jnp.float32),
            pltpu.VMEM((_BATCH,), jnp.int32),
            pltpu.VMEM((_BATCH,), jnp.int32),
            pltpu.VMEM((_BATCH,), jnp.float32),
            pltpu.VMEM((_BATCH,), jnp.float32),
            pltpu.SemaphoreType.DMA,
            pltpu.SemaphoreType.DMA,
            pltpu.SemaphoreType.DMA,
            pltpu.SemaphoreType.DMA,
            pltpu.SemaphoreType.DMA,
        ],
    )(xT, tt)
    return jnp.transpose(outT)


# speedup vs baseline: 4.7986x; 1.0325x over previous
import jax
import jax.numpy as jnp
from jax import lax
from jax.experimental import pallas as pl
from jax.experimental.pallas import tpu as pltpu
from jax.experimental.pallas import tpu_sc as plsc

_N_FIELDS = 26
_VOCAB = 100000
_EMB = 64
_BATCH = 4096
_NC, _NS, _L = 2, 16, 16
_NW = _NC * _NS
_NTASK = _N_FIELDS * _EMB
_TPW = _NTASK // _NW
_W0 = 50048
_W1 = _VOCAB - _W0
_NG = _BATCH // _L


def _tile_body(xT_hbm, tt_hbm, outT_hbm, tvA, tvB, xf, ov0, ov1,
               tsemA, tsemB, ssem0, ssem1):
    wid = lax.axis_index("s") * _NC + lax.axis_index("c")
    base = wid * _TPW
    i_first = base // _EMB

    pltpu.sync_copy(xT_hbm.at[i_first], xf.at[pl.ds(0, _BATCH)])
    pltpu.sync_copy(xT_hbm.at[(base + _TPW - 1) // _EMB],
                    xf.at[pl.ds(_BATCH, _BATCH)])

    def stream_half(t, half):
        i = t // _EMB
        c = t % _EMB
        if half == 0:
            return pltpu.make_async_copy(
                tt_hbm.at[i, c, pl.ds(0, _W0)], tvA, tsemA)
        return pltpu.make_async_copy(
            tt_hbm.at[i, c, pl.ds(_W0, _W1)], tvB, tsemB)

    def gather_half(t, ov, half):
        xoff = (t // _EMB - i_first) * _BATCH

        def grp(g, _):
            for u in range(16):
                off = (g * 16 + u) * _L
                idx = xf[pl.ds(xoff + off, _L)]
                if half == 0:
                    m = idx < _W0
                    v = plsc.load_gather(tvA, [idx], mask=m)
                    ov[pl.ds(off, _L)] = v
                else:
                    m = idx >= _W0
                    v = plsc.load_gather(tvB, [idx - _W0], mask=m)
                    ov[pl.ds(off, _L)] = jnp.where(m, v, ov[pl.ds(off, _L)])
            return 0

        lax.fori_loop(0, _NG // 16, grp, 0)

    def do_task(t, ov, ssem, nxt, nxt_guard):
        def refill(half):
            if nxt_guard is None:
                stream_half(nxt, half).start()
            else:
                @pl.when(nxt_guard)
                def _():
                    stream_half(nxt, half).start()

        stream_half(t, 0).wait()
        gather_half(t, ov, 0)
        refill(0)
        stream_half(t, 1).wait()
        gather_half(t, ov, 1)
        refill(1)
        pltpu.make_async_copy(ov, outT_hbm.at[t], ssem).start()

    stream_half(base, 0).start()
    stream_half(base, 1).start()

    def body(j, _):
        t0 = base + 2 * j

        @pl.when(j >= 1)
        def _():
            pltpu.make_async_copy(ov0, outT_hbm.at[t0], ssem0).wait()
        do_task(t0, ov0, ssem0, t0 + 1, None)

        @pl.when(j >= 1)
        def _():
            pltpu.make_async_copy(ov1, outT_hbm.at[t0], ssem1).wait()
        do_task(t0 + 1, ov1, ssem1, t0 + 2, j <= _TPW // 2 - 2)
        return 0

    lax.fori_loop(0, _TPW // 2, body, 0)
    pltpu.make_async_copy(ov0, outT_hbm.at[base], ssem0).wait()
    pltpu.make_async_copy(ov1, outT_hbm.at[base], ssem1).wait()


def kernel(x, tables):
    xT = jnp.transpose(x)
    tt = jnp.transpose(tables, (0, 2, 1))
    mesh = plsc.VectorSubcoreMesh(core_axis_name="c", subcore_axis_name="s")
    outT = pl.kernel(
        _tile_body,
        out_type=jax.ShapeDtypeStruct((_NTASK, _BATCH), jnp.float32),
        mesh=mesh,
        compiler_params=pltpu.CompilerParams(needs_layout_passes=False),
        scratch_types=[
            pltpu.VMEM((_W0,), jnp.float32),
            pltpu.VMEM((_W1,), jnp.float32),
            pltpu.VMEM((2 * _BATCH,), jnp.int32),
            pltpu.VMEM((_BATCH,), jnp.float32),
            pltpu.VMEM((_BATCH,), jnp.float32),
            pltpu.SemaphoreType.DMA,
            pltpu.SemaphoreType.DMA,
            pltpu.SemaphoreType.DMA,
            pltpu.SemaphoreType.DMA,
        ],
    )(xT, tt)
    return jnp.transpose(outT)
